# baseline (speedup 1.0000x reference)
import functools

import jax
import jax.numpy as jnp
from jax import lax
from jax.experimental import pallas as pl
from jax.experimental.pallas import tpu as pltpu
from jax.experimental.pallas import tpu_sc as plsc

B, T, F = 1024, 50, 26
VOCAB, D = 1000, 16
L = 16
NU = T * F
NW = 32
BASE_CNT = NU // NW
EXTRA = NU - BASE_CNT * NW
NG = B // L


def _make_gather():
    mesh = plsc.VectorSubcoreMesh(core_axis_name="c", subcore_axis_name="s")

    @functools.partial(
        pl.kernel,
        mesh=mesh,
        out_type=jax.ShapeDtypeStruct((T, F * D, B), jnp.float32),
        scratch_types=[
            pltpu.VMEM((VOCAB * D,), jnp.float32),
            pltpu.VMEM((B,), jnp.int32),
            pltpu.VMEM((B,), jnp.int32),
            pltpu.VMEM((D, B), jnp.float32),
            pltpu.VMEM((D, B), jnp.float32),
            pltpu.SemaphoreType.DMA,
            pltpu.SemaphoreType.DMA,
            pltpu.SemaphoreType.DMA,
            pltpu.SemaphoreType.DMA,
            pltpu.SemaphoreType.DMA,
        ],
        compiler_params=pltpu.CompilerParams(use_tc_tiling_on_sc=True,
                                             needs_layout_passes=False),
    )
    def gather_kernel(x_hbm, table_hbm, out_hbm, table_v, xin0, xin1,
                      slab0, slab1, tsem, isem0, isem1, wsem0, wsem1):
        wid = lax.axis_index("s") * 2 + lax.axis_index("c")
        start = wid * BASE_CNT + jnp.minimum(wid, EXTRA)
        count = BASE_CNT + jnp.where(wid < EXTRA, 1, 0)
        xins = (xin0, xin1)
        slabs = (slab0, slab1)
        isems = (isem0, isem1)
        wsems = (wsem0, wsem1)

        def unit_tf(j):
            m = start + j
            return m // F, lax.rem(m, F)

        def stage_in(j, u):
            t, f = unit_tf(j)
            return pltpu.async_copy(x_hbm.at[f, t], xins[u], isems[u])

        def wait_in(j, u):
            t, f = unit_tf(j)
            pltpu.make_async_copy(x_hbm.at[f, t], xins[u], isems[u]).wait()

        def stage_out(j, u):
            t, f = unit_tf(j)
            return pltpu.async_copy(
                slabs[u], out_hbm.at[t, pl.ds(f * D, D)], wsems[u])

        def wait_out(j, u):
            t, f = unit_tf(j)
            pltpu.make_async_copy(
                slabs[u], out_hbm.at[t, pl.ds(f * D, D)], wsems[u]).wait()

        tcopy = pltpu.async_copy(table_hbm, table_v, tsem)
        for u in range(2):
            stage_in(u, u)
        tcopy.wait()

        def gather_unit(xin, slab):
            @plsc.parallel_loop(0, NG, unroll=2)
            def grp(g):
                a0 = xin[pl.ds(g * L, L)] * D
                for c in range(D):
                    slab[c, pl.ds(g * L, L)] = (
                        plsc.load_gather(table_v, [a0 + c]))

        for u in range(2):
            wait_in(u, u)
            gather_unit(xins[u], slabs[u])
            stage_in(u + 2, u)
            stage_out(u, u)

        def pair(i, carry):
            for u in range(2):
                j = 2 * i + u
                wait_out(j, u)
                wait_in(j, u)
                gather_unit(xins[u], slabs[u])
                stage_in(lax.rem(j + 2, count), u)
                stage_out(j, u)
            return carry

        lax.fori_loop(1, count // 2, pair, 0)

        @pl.when(lax.rem(count, 2) == 1)
        def _tail():
            j = count - 1
            wait_out(j, 0)
            wait_in(j, 0)
            gather_unit(xins[0], slabs[0])
            stage_in(0, 0)
            stage_out(j, 0)

        for u in range(2):
            wait_out(u, u)
            wait_in(u, u)

    return gather_kernel


_gather = _make_gather()


def kernel(x, vec_of_char):
    xt = jnp.transpose(x, (2, 1, 0))
    out_t = _gather(xt, vec_of_char.reshape(VOCAB * D))
    return jnp.transpose(out_t, (2, 0, 1))


# speedup vs baseline: 1.0248x; 1.0248x over previous
import functools

import jax
import jax.numpy as jnp
from jax import lax
from jax.experimental import pallas as pl
from jax.experimental.pallas import tpu as pltpu
from jax.experimental.pallas import tpu_sc as plsc

B, T, F = 1024, 50, 26
VOCAB, D = 1000, 16
L = 16
FH = F // 2
TH = T // 2
KH = FH * D
NG = 128 // L


def _make_gather():
    mesh = plsc.VectorSubcoreMesh(core_axis_name="c", subcore_axis_name="s")

    @functools.partial(
        pl.kernel,
        mesh=mesh,
        out_type=jax.ShapeDtypeStruct((T, F * D, B), jnp.float32),
        scratch_types=[
            pltpu.VMEM((VOCAB * D,), jnp.float32),
            pltpu.VMEM((FH, 128), jnp.int32),
            pltpu.VMEM((FH, 128), jnp.int32),
            pltpu.VMEM((KH, 128), jnp.float32),
            pltpu.VMEM((KH, 128), jnp.float32),
            pltpu.SemaphoreType.DMA,
            pltpu.SemaphoreType.DMA,
            pltpu.SemaphoreType.DMA,
            pltpu.SemaphoreType.DMA,
            pltpu.SemaphoreType.DMA,
        ],
        compiler_params=pltpu.CompilerParams(use_tc_tiling_on_sc=True,
                                             needs_layout_passes=False),
    )
    def gather_kernel(x_hbm, table_hbm, out_hbm, table_v, xin0, xin1,
                      slab0, slab1, tsem, isem0, isem1, wsem0, wsem1):
        wid = lax.axis_index("s") * 2 + lax.axis_index("c")
        bt = lax.rem(wid, 8)
        kh = lax.rem(wid // 8, 2)
        tg = wid // 16
        bq = bt * 128
        f0 = kh * FH
        k0 = kh * KH
        t0 = tg * TH
        xins = (xin0, xin1)
        slabs = (slab0, slab1)
        isems = (isem0, isem1)
        wsems = (wsem0, wsem1)

        def stage_in(t, u):
            return pltpu.async_copy(
                x_hbm.at[pl.ds(f0, FH), t, pl.ds(bq, 128)], xins[u],
                isems[u])

        def stage_out(t, u):
            return pltpu.async_copy(
                slabs[u], out_hbm.at[t, pl.ds(k0, KH), pl.ds(bq, 128)],
                wsems[u])

        tcopy = pltpu.async_copy(table_hbm, table_v, tsem)
        icopies = [stage_in(t0 + u, u) for u in range(2)]
        tcopy.wait()

        def gather_t(xin, slab):
            @plsc.parallel_loop(0, FH * NG, unroll=2)
            def unit(m):
                f = m >> 3
                g = lax.rem(m, NG)
                a0 = xin[f, pl.ds(g * L, L)] * D
                for c in range(D):
                    slab[f * D + c, pl.ds(g * L, L)] = (
                        plsc.load_gather(table_v, [a0 + c]))

        for u in range(2):
            icopies[u].wait()
            gather_t(xins[u], slabs[u])
            stage_in(t0 + u + 2, u)
            stage_out(t0 + u, u)

        def pair(i, carry):
            for u in range(2):
                t = t0 + 2 * i + u
                pltpu.make_async_copy(
                    slabs[u], out_hbm.at[t, pl.ds(k0, KH), pl.ds(bq, 128)],
                    wsems[u]).wait()
                pltpu.make_async_copy(
                    x_hbm.at[pl.ds(f0, FH), t, pl.ds(bq, 128)], xins[u],
                    isems[u]).wait()
                gather_t(xins[u], slabs[u])
                stage_in(t0 + lax.rem(2 * i + u + 2, TH), u)
                stage_out(t, u)
            return carry

        lax.fori_loop(1, TH // 2, pair, 0)

        tl = t0 + TH - 1
        pltpu.make_async_copy(
            slabs[0], out_hbm.at[tl, pl.ds(k0, KH), pl.ds(bq, 128)],
            wsems[0]).wait()
        pltpu.make_async_copy(
            x_hbm.at[pl.ds(f0, FH), tl, pl.ds(bq, 128)], xins[0],
            isems[0]).wait()
        gather_t(xins[0], slabs[0])
        stage_out(tl, 0)

        for u in range(2):
            pltpu.make_async_copy(
                slabs[u], out_hbm.at[t0, pl.ds(k0, KH), pl.ds(bq, 128)],
                wsems[u]).wait()
        pltpu.make_async_copy(
            x_hbm.at[pl.ds(f0, FH), t0, pl.ds(bq, 128)], xins[1],
            isems[1]).wait()

    return gather_kernel


_gather = _make_gather()


def kernel(x, vec_of_char):
    xt = jnp.transpose(x, (2, 1, 0))
    out_t = _gather(xt, vec_of_char.reshape(VOCAB * D))
    return jnp.transpose(out_t, (2, 0, 1))
